# baseline (speedup 1.0000x reference)
import functools

import jax
import jax.numpy as jnp
from jax import lax
from jax.experimental import pallas as pl
from jax.experimental.pallas import tpu as pltpu
from jax.experimental.pallas import tpu_sc as plsc

_B, _D, _V = 8, 64, 1_000_000

_TILE = 32768
_NW = 32
_CHUNK = 512
_V_SC = 262144
_CPW = _V_SC // _NW
_GRP = 4

_TC_BLK0 = _V_SC // _TILE


def _tc_body(state_ref, w_ref, out_ref):
    out_ref[...] = lax.dot_general(
        state_ref[...],
        w_ref[...],
        (((1,), (0,)), ((), ())),
        preferred_element_type=jnp.float32,
    )


def _tc_matmul(state, W):
    grid = (pl.cdiv(_V - _V_SC, _TILE),)
    return pl.pallas_call(
        _tc_body,
        grid=grid,
        in_specs=[
            pl.BlockSpec((_B, _D), lambda i: (0, 0)),
            pl.BlockSpec((_D, _TILE), lambda i: (0, i + _TC_BLK0)),
        ],
        out_specs=pl.BlockSpec((_B, _TILE), lambda i: (0, i + _TC_BLK0)),
        out_shape=jax.ShapeDtypeStruct((_B, _V), jnp.float32),
        compiler_params=pltpu.CompilerParams(
            dimension_semantics=("arbitrary",),
        ),
    )(state, W)


@functools.partial(
    pl.kernel,
    out_type=jax.ShapeDtypeStruct((_B, _V_SC), jnp.float32),
    mesh=plsc.VectorSubcoreMesh(core_axis_name="c", subcore_axis_name="s"),
    scratch_types=[
        pltpu.VMEM((_B, _D), jnp.float32),
        pltpu.VMEM((_D, _CHUNK), jnp.float32),
        pltpu.VMEM((_B, _CHUNK), jnp.float32),
    ],
)
def _sc_matmul(state_hbm, w_hbm, out_hbm, state_v, wbuf, obuf):
    nc = 2
    wid = lax.axis_index("s") * nc + lax.axis_index("c")
    base = wid * _CPW
    pltpu.sync_copy(state_hbm, state_v)

    def chunk_body(ci, carry):
        off = base + ci * _CHUNK
        pltpu.sync_copy(w_hbm.at[:, pl.ds(off, _CHUNK)], wbuf)

        def blk_body(bi, inner):
            c0 = bi * (16 * _GRP)
            zero = jnp.zeros((16,), jnp.float32)
            accs0 = [[zero for _ in range(_GRP)] for _ in range(_B)]

            def k_step(j, accs):
                svs = [state_v[b, pl.ds(j * 16, 16)] for b in range(_B)]
                for u in range(16):
                    k = j * 16 + u
                    wv = [wbuf[k, pl.ds(c0 + g * 16, 16)] for g in range(_GRP)]
                    for b in range(_B):
                        s = svs[b][u]
                        for g in range(_GRP):
                            accs[b][g] = accs[b][g] + s * wv[g]
                return accs

            accs = lax.fori_loop(0, _D // 16, k_step, accs0)
            for b in range(_B):
                for g in range(_GRP):
                    obuf[b, pl.ds(c0 + g * 16, 16)] = accs[b][g]
            return inner

        lax.fori_loop(0, _CHUNK // (16 * _GRP), blk_body, 0)
        pltpu.sync_copy(obuf, out_hbm.at[:, pl.ds(off, _CHUNK)])
        return carry

    lax.fori_loop(0, _CPW // _CHUNK, chunk_body, 0)


@jax.jit
def kernel(state, W):
    tc_out = _tc_matmul(state, W)
    sc_out = _sc_matmul(state, W)
    return lax.dynamic_update_slice(tc_out, sc_out, (0, 0))
